# Initial kernel scaffold; baseline (speedup 1.0000x reference)
import jax
import jax.numpy as jnp
from jax.experimental import pallas as pl


def kernel(xyz1, xyz2, points1, points2, W1, b1, W2, b2):
    raise NotImplementedError("write your pallas kernel here")


# speedup vs baseline: 22.6937x; 22.6937x over previous
import functools

import jax
import jax.numpy as jnp
import numpy as np
from jax.experimental import pallas as pl
from jax.experimental.pallas import tpu as pltpu


def _fp_body(xyz1_ref, xyz2_ref, p1_ref, p2_ref, w1a_ref, w1b_ref, b1_ref,
             w2_ref, b2_ref, out_ref, *, tn, s):
    q = xyz1_ref[0]
    k = xyz2_ref[0]
    q0, q1, q2 = q[0], q[1], q[2]
    k0, k1, k2 = k[0], k[1], k[2]
    sq1 = q0 * q0 + q1 * q1 + q2 * q2
    sq2 = k0 * k0 + k1 * k1 + k2 * k2
    qk = jax.lax.dot_general(q.astype(jnp.bfloat16), k.astype(jnp.bfloat16),
                             (((0,), (0,)), ((), ())),
                             preferred_element_type=jnp.float32)
    d = sq1[:, None] + sq2[None, :] - 2.0 * qk

    iota = jax.lax.broadcasted_iota(jnp.int32, (tn, s), 1)
    inf = jnp.float32(np.inf)
    dcur = d
    recips = []
    onehots = []
    for _ in range(3):
        mv = jnp.min(dcur, axis=1)
        eq = dcur == mv[:, None]
        idx = jnp.min(jnp.where(eq, iota, s), axis=1)
        sel = iota == idx[:, None]
        recips.append(1.0 / (mv + 1e-8))
        onehots.append(sel)
        dcur = jnp.where(sel, inf, dcur)
    norm = recips[0] + recips[1] + recips[2]
    wm = jnp.zeros((tn, s), jnp.float32)
    for j in range(3):
        wm = wm + jnp.where(onehots[j], (recips[j] / norm)[:, None], 0.0)

    p2 = p2_ref[0]
    interp = jax.lax.dot_general(p2, wm, (((1,), (1,)), ((), ())),
                                 precision=jax.lax.Precision.HIGHEST,
                                 preferred_element_type=jnp.float32)

    bf = jnp.bfloat16
    p1 = p1_ref[0]
    h = jax.lax.dot_general(w1a_ref[...].astype(bf), p1.astype(bf),
                            (((1,), (0,)), ((), ())),
                            preferred_element_type=jnp.float32)
    h = h + jax.lax.dot_general(w1b_ref[...].astype(bf), interp.astype(bf),
                                (((1,), (0,)), ((), ())),
                                preferred_element_type=jnp.float32)
    h = jnp.maximum(h + b1_ref[...][:, 0][:, None], 0.0)
    h2 = jax.lax.dot_general(w2_ref[...].astype(bf), h.astype(bf),
                             (((1,), (0,)), ((), ())),
                             preferred_element_type=jnp.float32)
    h2 = jnp.maximum(h2 + b2_ref[...][:, 0][:, None], 0.0)
    out_ref[0] = h2


def kernel(xyz1, xyz2, points1, points2, W1, b1, W2, b2):
    B, _, N = xyz1.shape
    S = xyz2.shape[2]
    C1 = points1.shape[1]
    C2 = points2.shape[1]
    O1 = W1.shape[0]
    O2 = W2.shape[0]
    TN = min(512, N)

    w1a = W1[:, :C1]
    w1b = W1[:, C1:]
    b1c = b1[:, None]
    b2c = b2[:, None]

    grid = (B, N // TN)
    body = functools.partial(_fp_body, tn=TN, s=S)
    out = pl.pallas_call(
        body,
        grid=grid,
        in_specs=[
            pl.BlockSpec((1, 3, TN), lambda b, n: (b, 0, n)),
            pl.BlockSpec((1, 3, S), lambda b, n: (b, 0, 0)),
            pl.BlockSpec((1, C1, TN), lambda b, n: (b, 0, n)),
            pl.BlockSpec((1, C2, S), lambda b, n: (b, 0, 0)),
            pl.BlockSpec((O1, C1), lambda b, n: (0, 0)),
            pl.BlockSpec((O1, C2), lambda b, n: (0, 0)),
            pl.BlockSpec((O1, 1), lambda b, n: (0, 0)),
            pl.BlockSpec((O2, O1), lambda b, n: (0, 0)),
            pl.BlockSpec((O2, 1), lambda b, n: (0, 0)),
        ],
        out_specs=pl.BlockSpec((1, O2, TN), lambda b, n: (b, 0, n)),
        out_shape=jax.ShapeDtypeStruct((B, O2, N), jnp.float32),
        compiler_params=pltpu.CompilerParams(
            dimension_semantics=("parallel", "arbitrary"),
        ),
    )(xyz1, xyz2, points1, points2, w1a, w1b, b1c, W2, b2c)
    return out
